# baseline (speedup 1.0000x reference)
import functools

import jax
import jax.numpy as jnp
from jax import lax
from jax.experimental import pallas as pl
from jax.experimental.pallas import tpu as pltpu
from jax.experimental.pallas import tpu_sc as plsc

USER = 5000
ITEM = 5000
N_NODES = USER + ITEM
LATDIM = 128
N_EDGES = 320000

NC = 2
NS = 16
DH = LATDIM // NC
CHUNK = 128
NBUF = 4
LOOK = 2
IBUF = 8
E_PAD = ((N_EDGES + NS * CHUNK * IBUF - 1)
         // (NS * CHUNK * IBUF)) * (NS * CHUNK * IBUF)
PER_TILE = E_PAD // NS
NCHUNK = PER_TILE // CHUNK
N_PAD = 10240
RB = 128
ROWS_PER_TILE = N_PAD // NS
NRB = ROWS_PER_TILE // RB
ZB = 32

_mesh = plsc.VectorSubcoreMesh(core_axis_name="c", subcore_axis_name="s",
                               num_cores=NC, num_subcores=NS)


@functools.partial(
    pl.kernel,
    out_type=jax.ShapeDtypeStruct((NC * N_PAD, DH), jnp.float32),
    mesh=_mesh,
    scratch_types=(
        [pltpu.MemorySpace.HBM((NC * N_PAD, DH), jnp.float32)]
        + [pltpu.VMEM_SHARED((N_PAD, DH), jnp.float32)]
        + [pltpu.VMEM((CHUNK,), jnp.int32)] * IBUF
        + [pltpu.VMEM((CHUNK,), jnp.int32)] * IBUF
        + [pltpu.VMEM((CHUNK,), jnp.float32)] * IBUF
        + [pltpu.VMEM((CHUNK, DH), jnp.float32)] * NBUF
        + [pltpu.VMEM((ZB, DH), jnp.float32)]
        + [pltpu.SemaphoreType.DMA] * IBUF
        + [pltpu.SemaphoreType.DMA] * NBUF
        + [pltpu.SemaphoreType.DMA] * NBUF
    ),
    compiler_params=pltpu.CompilerParams(use_tc_tiling_on_sc=False),
)
def _gcn_sc(col_hbm, row_hbm, val_hbm, emb_hbm, out_hbm, tbl1_hbm,
            acc_sh, *rest):
    colb = rest[0:IBUF]
    rowb = rest[IBUF:2 * IBUF]
    valb = rest[2 * IBUF:3 * IBUF]
    k = 3 * IBUF
    bufs = rest[k:k + NBUF]
    zblk = rest[k + NBUF]
    k = k + NBUF + 1
    isems = rest[k:k + IBUF]
    gsems = rest[k + IBUF:k + IBUF + NBUF]
    ssems = rest[k + IBUF + NBUF:k + IBUF + 2 * NBUF]

    c = lax.axis_index("c")
    s = lax.axis_index("s")
    ebase = s * PER_TILE
    rbase = s * ROWS_PER_TILE
    coff = c * N_PAD

    def zero_body(r, _):
        for j in range(DH // 16):
            zblk[r, pl.ds(j * 16, 16)] = jnp.zeros((16,), jnp.float32)
        return _
    lax.fori_loop(0, ZB, zero_body, None)
    for b in range(NRB):
        r0 = rbase + b * RB
        for z in range(RB // ZB):
            pltpu.sync_copy(zblk, acc_sh.at[pl.ds(r0 + z * ZB, ZB), :])
    plsc.subcore_barrier()

    def spmm(src_hbm):
        def fire_idx(i, g):
            off = ebase + g * CHUNK
            pltpu.async_copy(col_hbm.at[c, pl.ds(off, CHUNK)], colb[i],
                             isems[i])
            pltpu.async_copy(row_hbm.at[pl.ds(off, CHUNK)], rowb[i], isems[i])
            pltpu.async_copy(val_hbm.at[pl.ds(off, CHUNK)], valb[i], isems[i])

        def wait_idx(i):
            pltpu.make_async_copy(col_hbm.at[0, pl.ds(0, CHUNK)], colb[i],
                                  isems[i]).wait()
            pltpu.make_async_copy(row_hbm.at[pl.ds(0, CHUNK)], rowb[i],
                                  isems[i]).wait()
            pltpu.make_async_copy(val_hbm.at[pl.ds(0, CHUNK)], valb[i],
                                  isems[i]).wait()

        def fire_gather(b, i):
            pltpu.async_copy(src_hbm.at[colb[i]], bufs[b], gsems[b])

        def wait_gather(b, i):
            pltpu.make_async_copy(src_hbm.at[colb[i]], bufs[b],
                                  gsems[b]).wait()

        def fire_scatter(b, i):
            pltpu.async_copy(bufs[b], acc_sh.at[rowb[i]], ssems[b], add=True)

        def wait_scatter(b, i):
            pltpu.make_async_copy(bufs[b], acc_sh.at[rowb[i]],
                                  ssems[b]).wait()

        def scale(b, i):
            buf = bufs[b]
            def grp(g2, _):
                e0 = g2 * 16
                vals16 = valb[i][pl.ds(e0, 16)]
                for kk in range(16):
                    v = vals16[kk]
                    for j in range(DH // 16):
                        buf[e0 + kk, pl.ds(j * 16, 16)] = (
                            buf[e0 + kk, pl.ds(j * 16, 16)] * v)
                return _
            lax.fori_loop(0, CHUNK // 16, grp, None)

        for g in range(6):
            fire_idx(g % IBUF, g)
        for g in range(LOOK):
            wait_idx(g)
            fire_gather(g % NBUF, g)

        def pipe_body(t, _):
            for sl in range(IBUF):
                g = IBUF * t + sl
                b = sl % NBUF
                @pl.when(jnp.logical_and(g >= LOOK, g + LOOK < NCHUNK))
                def _():
                    wait_scatter((sl + LOOK) % NBUF, (sl + LOOK) % IBUF)

                @pl.when(g + LOOK < NCHUNK)
                def _():
                    wait_idx((sl + LOOK) % IBUF)
                    fire_gather((sl + LOOK) % NBUF, (sl + LOOK) % IBUF)

                @pl.when(g + 6 < NCHUNK)
                def _():
                    fire_idx((sl + 6) % IBUF, g + 6)

                wait_gather(b, sl)
                scale(b, sl)
                fire_scatter(b, sl)
            return _
        lax.fori_loop(0, NCHUNK // IBUF, pipe_body, None)
        for g in range(NCHUNK - NBUF, NCHUNK):
            wait_scatter(g % NBUF, g % IBUF)

    spmm(emb_hbm)
    plsc.subcore_barrier()
    for b in range(NRB):
        r0 = rbase + b * RB
        pltpu.sync_copy(acc_sh.at[pl.ds(r0, RB), :], bufs[0])
        pltpu.sync_copy(bufs[0], tbl1_hbm.at[pl.ds(coff + r0, RB), :])
        for z in range(RB // ZB):
            pltpu.sync_copy(zblk, acc_sh.at[pl.ds(r0 + z * ZB, ZB), :])
    plsc.subcore_barrier()
    spmm(tbl1_hbm)
    plsc.subcore_barrier()
    for b in range(NRB):
        r0 = rbase + b * RB
        pltpu.sync_copy(acc_sh.at[pl.ds(r0, RB), :], bufs[0])
        pltpu.sync_copy(emb_hbm.at[pl.ds(coff + r0, RB), :], bufs[1])
        pltpu.sync_copy(tbl1_hbm.at[pl.ds(coff + r0, RB), :], bufs[2])

        def add_body(r, _):
            for j in range(DH // 16):
                bufs[0][r, pl.ds(j * 16, 16)] = (
                    bufs[0][r, pl.ds(j * 16, 16)]
                    + bufs[1][r, pl.ds(j * 16, 16)]
                    + bufs[2][r, pl.ds(j * 16, 16)])
            return _
        lax.fori_loop(0, RB, add_body, None)
        pltpu.sync_copy(bufs[0], out_hbm.at[pl.ds(coff + r0, RB), :])


@jax.jit
def kernel(adj_indices, adj_values, uEmbeds, iEmbeds):
    row = adj_indices[0].astype(jnp.int32)
    col = adj_indices[1].astype(jnp.int32)
    pad = E_PAD - N_EDGES
    row = jnp.pad(row, (0, pad))
    col = jnp.pad(col, (0, pad))
    val = jnp.pad(adj_values, (0, pad))

    embeds = jnp.concatenate([uEmbeds, iEmbeds], axis=0)
    emb2 = jnp.zeros((NC * N_PAD, DH), jnp.float32)
    emb2 = emb2.at[:N_NODES].set(embeds[:, :DH])
    emb2 = emb2.at[N_PAD:N_PAD + N_NODES].set(embeds[:, DH:])

    col2 = jnp.stack([col, col + N_PAD])
    out2 = _gcn_sc(col2, row, val, emb2)
    total = jnp.concatenate(
        [out2[:N_NODES], out2[N_PAD:N_PAD + N_NODES]], axis=1)
    return (total[:USER], total[USER:])


# speedup vs baseline: 1.0458x; 1.0458x over previous
import functools

import jax
import jax.numpy as jnp
from jax import lax
from jax.experimental import pallas as pl
from jax.experimental.pallas import tpu as pltpu
from jax.experimental.pallas import tpu_sc as plsc

USER = 5000
ITEM = 5000
N_NODES = USER + ITEM
LATDIM = 128
N_EDGES = 320000

NC = 2
NS = 16
DH = LATDIM // NC
CHUNK = 256
NBUF = 4
LOOK = 2
IBUF = 8
E_PAD = ((N_EDGES + NS * CHUNK * IBUF - 1)
         // (NS * CHUNK * IBUF)) * (NS * CHUNK * IBUF)
PER_TILE = E_PAD // NS
NCHUNK = PER_TILE // CHUNK
N_PAD = 10240
RB = 128
ROWS_PER_TILE = N_PAD // NS
NRB = ROWS_PER_TILE // RB
ZB = 32

_mesh = plsc.VectorSubcoreMesh(core_axis_name="c", subcore_axis_name="s",
                               num_cores=NC, num_subcores=NS)


@functools.partial(
    pl.kernel,
    out_type=jax.ShapeDtypeStruct((NC * N_PAD, DH), jnp.float32),
    mesh=_mesh,
    scratch_types=(
        [pltpu.MemorySpace.HBM((NC * N_PAD, DH), jnp.float32)]
        + [pltpu.VMEM_SHARED((N_PAD, DH), jnp.float32)]
        + [pltpu.VMEM((2, 128), jnp.int32)] * IBUF
        + [pltpu.VMEM((2, 128), jnp.int32)] * IBUF
        + [pltpu.VMEM((CHUNK,), jnp.float32)] * IBUF
        + [pltpu.VMEM((CHUNK, DH), jnp.float32)] * NBUF
        + [pltpu.VMEM((ZB, DH), jnp.float32)]
        + [pltpu.SemaphoreType.DMA] * IBUF
        + [pltpu.SemaphoreType.DMA] * NBUF
        + [pltpu.SemaphoreType.DMA] * NBUF
    ),
    compiler_params=pltpu.CompilerParams(use_tc_tiling_on_sc=False),
)
def _gcn_sc(col_hbm, row_hbm, val_hbm, emb_hbm, out_hbm, tbl1_hbm,
            acc_sh, *rest):
    colb = rest[0:IBUF]
    rowb = rest[IBUF:2 * IBUF]
    valb = rest[2 * IBUF:3 * IBUF]
    k = 3 * IBUF
    bufs = rest[k:k + NBUF]
    zblk = rest[k + NBUF]
    k = k + NBUF + 1
    isems = rest[k:k + IBUF]
    gsems = rest[k + IBUF:k + IBUF + NBUF]
    ssems = rest[k + IBUF + NBUF:k + IBUF + 2 * NBUF]

    c = lax.axis_index("c")
    s = lax.axis_index("s")
    ebase = s * PER_TILE
    rbase = s * ROWS_PER_TILE
    coff = c * N_PAD

    def zero_body(r, _):
        for j in range(DH // 16):
            zblk[r, pl.ds(j * 16, 16)] = jnp.zeros((16,), jnp.float32)
        return _
    lax.fori_loop(0, ZB, zero_body, None)
    for b in range(NRB):
        r0 = rbase + b * RB
        for z in range(RB // ZB):
            pltpu.sync_copy(zblk, acc_sh.at[pl.ds(r0 + z * ZB, ZB), :])
    plsc.subcore_barrier()

    def spmm(src_hbm):
        def fire_idx(i, g):
            r128 = (ebase + g * CHUNK) // 128
            off = ebase + g * CHUNK
            pltpu.async_copy(col_hbm.at[c, pl.ds(r128, 2), :], colb[i],
                             isems[i])
            pltpu.async_copy(row_hbm.at[pl.ds(r128, 2), :], rowb[i], isems[i])
            pltpu.async_copy(val_hbm.at[pl.ds(off, CHUNK)], valb[i], isems[i])

        def wait_idx(i):
            pltpu.make_async_copy(col_hbm.at[0, pl.ds(0, 2), :], colb[i],
                                  isems[i]).wait()
            pltpu.make_async_copy(row_hbm.at[pl.ds(0, 2), :], rowb[i],
                                  isems[i]).wait()
            pltpu.make_async_copy(val_hbm.at[pl.ds(0, CHUNK)], valb[i],
                                  isems[i]).wait()

        def fire_gather(b, i):
            for h in range(2):
                pltpu.async_copy(src_hbm.at[colb[i].at[h]],
                                 bufs[b].at[pl.ds(h * 128, 128), :],
                                 gsems[b])

        def wait_gather(b, i):
            for h in range(2):
                pltpu.make_async_copy(src_hbm.at[colb[i].at[h]],
                                      bufs[b].at[pl.ds(h * 128, 128), :],
                                      gsems[b]).wait()

        def fire_scatter(b, i):
            for h in range(2):
                pltpu.async_copy(bufs[b].at[pl.ds(h * 128, 128), :],
                                 acc_sh.at[rowb[i].at[h]], ssems[b],
                                 add=True)

        def wait_scatter(b, i):
            for h in range(2):
                pltpu.make_async_copy(bufs[b].at[pl.ds(h * 128, 128), :],
                                      acc_sh.at[rowb[i].at[0]],
                                      ssems[b]).wait()

        def scale(b, i):
            buf = bufs[b]
            def grp(g2, _):
                e0 = g2 * 16
                vals16 = valb[i][pl.ds(e0, 16)]
                for kk in range(16):
                    v = vals16[kk]
                    for j in range(DH // 16):
                        buf[e0 + kk, pl.ds(j * 16, 16)] = (
                            buf[e0 + kk, pl.ds(j * 16, 16)] * v)
                return _
            lax.fori_loop(0, CHUNK // 16, grp, None)

        for g in range(6):
            fire_idx(g % IBUF, g)
        for g in range(LOOK):
            wait_idx(g)
            fire_gather(g % NBUF, g)

        def pipe_body(t, _):
            for sl in range(IBUF):
                g = IBUF * t + sl
                b = sl % NBUF
                @pl.when(jnp.logical_and(g >= LOOK, g + LOOK < NCHUNK))
                def _():
                    wait_scatter((sl + LOOK) % NBUF, (sl + LOOK) % IBUF)

                @pl.when(g + LOOK < NCHUNK)
                def _():
                    wait_idx((sl + LOOK) % IBUF)
                    fire_gather((sl + LOOK) % NBUF, (sl + LOOK) % IBUF)

                @pl.when(g + 6 < NCHUNK)
                def _():
                    fire_idx((sl + 6) % IBUF, g + 6)

                wait_gather(b, sl)
                scale(b, sl)
                fire_scatter(b, sl)
            return _
        lax.fori_loop(0, NCHUNK // IBUF, pipe_body, None)
        for g in range(NCHUNK - NBUF, NCHUNK):
            wait_scatter(g % NBUF, g % IBUF)

    spmm(emb_hbm)
    plsc.subcore_barrier()
    for b in range(NRB):
        r0 = rbase + b * RB
        pltpu.sync_copy(acc_sh.at[pl.ds(r0, RB), :],
                        bufs[0].at[pl.ds(0, RB), :])
        pltpu.sync_copy(bufs[0].at[pl.ds(0, RB), :],
                        tbl1_hbm.at[pl.ds(coff + r0, RB), :])
        for z in range(RB // ZB):
            pltpu.sync_copy(zblk, acc_sh.at[pl.ds(r0 + z * ZB, ZB), :])
    plsc.subcore_barrier()
    spmm(tbl1_hbm)
    plsc.subcore_barrier()
    for b in range(NRB):
        r0 = rbase + b * RB
        pltpu.sync_copy(acc_sh.at[pl.ds(r0, RB), :],
                        bufs[0].at[pl.ds(0, RB), :])
        pltpu.sync_copy(emb_hbm.at[pl.ds(coff + r0, RB), :],
                        bufs[1].at[pl.ds(0, RB), :])
        pltpu.sync_copy(tbl1_hbm.at[pl.ds(coff + r0, RB), :],
                        bufs[2].at[pl.ds(0, RB), :])

        def add_body(r, _):
            for j in range(DH // 16):
                bufs[0][r, pl.ds(j * 16, 16)] = (
                    bufs[0][r, pl.ds(j * 16, 16)]
                    + bufs[1][r, pl.ds(j * 16, 16)]
                    + bufs[2][r, pl.ds(j * 16, 16)])
            return _
        lax.fori_loop(0, RB, add_body, None)
        pltpu.sync_copy(bufs[0].at[pl.ds(0, RB), :],
                        out_hbm.at[pl.ds(coff + r0, RB), :])


@jax.jit
def kernel(adj_indices, adj_values, uEmbeds, iEmbeds):
    row = adj_indices[0].astype(jnp.int32)
    col = adj_indices[1].astype(jnp.int32)
    pad = E_PAD - N_EDGES
    row = jnp.pad(row, (0, pad))
    col = jnp.pad(col, (0, pad))
    val = jnp.pad(adj_values, (0, pad))

    embeds = jnp.concatenate([uEmbeds, iEmbeds], axis=0)
    emb2 = jnp.zeros((NC * N_PAD, DH), jnp.float32)
    emb2 = emb2.at[:N_NODES].set(embeds[:, :DH])
    emb2 = emb2.at[N_PAD:N_PAD + N_NODES].set(embeds[:, DH:])

    col2 = jnp.stack([col, col + N_PAD])
    col2 = col2.reshape(NC, E_PAD // 128, 128)
    row2 = row.reshape(E_PAD // 128, 128)
    out2 = _gcn_sc(col2, row2, val, emb2)
    total = jnp.concatenate(
        [out2[:N_NODES], out2[N_PAD:N_PAD + N_NODES]], axis=1)
    return (total[:USER], total[USER:])


# speedup vs baseline: 1.4071x; 1.3454x over previous
import functools

import jax
import jax.numpy as jnp
import numpy as np
from jax import lax
from jax.experimental import pallas as pl
from jax.experimental.pallas import tpu as pltpu
from jax.experimental.pallas import tpu_sc as plsc

USER = 5000
ITEM = 5000
N_NODES = USER + ITEM
LATDIM = 128
N_EDGES = 320000

NC = 2
NS = 16
DH = LATDIM // NC
CHUNK = 256
NBUF = 4
SBUF = 2
LOOK = 3
IBUF = 8
E_PAD = ((N_EDGES + NS * CHUNK * IBUF - 1)
         // (NS * CHUNK * IBUF)) * (NS * CHUNK * IBUF)
PER_TILE = E_PAD // NS
NCHUNK = PER_TILE // CHUNK
N_PAD = 10240
RB = 128
ROWS_PER_TILE = N_PAD // NS
NRB = ROWS_PER_TILE // RB
ZB = 32

_DIM_ORDER = np.zeros(DH, dtype=np.int32)
for _q in range(DH // 32):
    for _l in range(16):
        _DIM_ORDER[32 * _q + 2 * _l] = 32 * _q + _l
        _DIM_ORDER[32 * _q + 2 * _l + 1] = 32 * _q + 16 + _l

_mesh = plsc.VectorSubcoreMesh(core_axis_name="c", subcore_axis_name="s",
                               num_cores=NC, num_subcores=NS)


@functools.partial(
    pl.kernel,
    out_type=jax.ShapeDtypeStruct((NC * N_PAD, DH), jnp.float32),
    mesh=_mesh,
    scratch_types=(
        [pltpu.MemorySpace.HBM((NC * N_PAD, DH), jnp.bfloat16)]
        + [pltpu.MemorySpace.HBM((NC * N_PAD, DH), jnp.float32)]
        + [pltpu.VMEM_SHARED((N_PAD, DH), jnp.float32)]
        + [pltpu.VMEM((2, 128), jnp.int32)] * IBUF
        + [pltpu.VMEM((2, 128), jnp.int32)] * IBUF
        + [pltpu.VMEM((CHUNK,), jnp.float32)] * IBUF
        + [pltpu.VMEM((CHUNK, DH), jnp.bfloat16)] * NBUF
        + [pltpu.VMEM((CHUNK, DH), jnp.float32)] * SBUF
        + [pltpu.VMEM((ZB, DH), jnp.float32)]
        + [pltpu.SemaphoreType.DMA] * IBUF
        + [pltpu.SemaphoreType.DMA] * NBUF
        + [pltpu.SemaphoreType.DMA] * SBUF
    ),
    compiler_params=pltpu.CompilerParams(use_tc_tiling_on_sc=False,
                                         needs_layout_passes=False),
)
def _gcn_sc(col_hbm, row_hbm, val_hbm, embbf_hbm, emb_hbm, out_hbm,
            tblbf_hbm, tblf_hbm, acc_sh, *rest):
    colb = rest[0:IBUF]
    rowb = rest[IBUF:2 * IBUF]
    valb = rest[2 * IBUF:3 * IBUF]
    k = 3 * IBUF
    gbufs = rest[k:k + NBUF]
    sbufs = rest[k + NBUF:k + NBUF + SBUF]
    zblk = rest[k + NBUF + SBUF]
    k = k + NBUF + SBUF + 1
    isems = rest[k:k + IBUF]
    gsems = rest[k + IBUF:k + IBUF + NBUF]
    ssems = rest[k + IBUF + NBUF:k + IBUF + NBUF + SBUF]

    c = lax.axis_index("c")
    s = lax.axis_index("s")
    ebase = s * PER_TILE
    rbase = s * ROWS_PER_TILE
    coff = c * N_PAD

    def zero_body(r, _):
        for j in range(DH // 16):
            zblk[r, pl.ds(j * 16, 16)] = jnp.zeros((16,), jnp.float32)
        return _
    lax.fori_loop(0, ZB, zero_body, None)
    for b in range(NRB):
        r0 = rbase + b * RB
        for z in range(RB // ZB):
            pltpu.sync_copy(zblk, acc_sh.at[pl.ds(r0 + z * ZB, ZB), :])
    plsc.subcore_barrier()

    def spmm(src_hbm):
        def fire_idx(i, g):
            r128 = (ebase + g * CHUNK) // 128
            off = ebase + g * CHUNK
            pltpu.async_copy(col_hbm.at[c, pl.ds(r128, 2), :], colb[i],
                             isems[i])
            pltpu.async_copy(row_hbm.at[pl.ds(r128, 2), :], rowb[i], isems[i])
            pltpu.async_copy(val_hbm.at[pl.ds(off, CHUNK)], valb[i], isems[i])

        def wait_idx(i):
            pltpu.make_async_copy(col_hbm.at[0, pl.ds(0, 2), :], colb[i],
                                  isems[i]).wait()
            pltpu.make_async_copy(row_hbm.at[pl.ds(0, 2), :], rowb[i],
                                  isems[i]).wait()
            pltpu.make_async_copy(val_hbm.at[pl.ds(0, CHUNK)], valb[i],
                                  isems[i]).wait()

        def fire_gather(b, i):
            for h in range(2):
                pltpu.async_copy(src_hbm.at[colb[i].at[h]],
                                 gbufs[b].at[pl.ds(h * 128, 128), :],
                                 gsems[b])

        def wait_gather(b, i):
            for h in range(2):
                pltpu.make_async_copy(src_hbm.at[colb[i].at[h]],
                                      gbufs[b].at[pl.ds(h * 128, 128), :],
                                      gsems[b]).wait()

        def fire_scatter(sb, i):
            for h in range(2):
                pltpu.async_copy(sbufs[sb].at[pl.ds(h * 128, 128), :],
                                 acc_sh.at[rowb[i].at[h]], ssems[sb],
                                 add=True)

        def wait_scatter(sb, i):
            for h in range(2):
                pltpu.make_async_copy(sbufs[sb].at[pl.ds(h * 128, 128), :],
                                      acc_sh.at[rowb[i].at[0]],
                                      ssems[sb]).wait()

        def scale(b, sb, i):
            gbuf = gbufs[b]
            sbuf = sbufs[sb]
            def grp(g2, _):
                e0 = g2 * 16
                vals16 = valb[i][pl.ds(e0, 16)]
                for kk in range(16):
                    v = vals16[kk]
                    for q in range(DH // 32):
                        half = gbuf[e0 + kk, pl.ds(q * 32, 32)]
                        a, bb = plsc.unpack(
                            half, format=plsc.PackFormat.INTERLEAVED)
                        sbuf[e0 + kk, pl.ds(q * 32, 16)] = a * v
                        sbuf[e0 + kk, pl.ds(q * 32 + 16, 16)] = bb * v
                return _
            lax.fori_loop(0, CHUNK // 16, grp, None)

        for g in range(6):
            fire_idx(g % IBUF, g)
        for g in range(LOOK):
            wait_idx(g)
            fire_gather(g % NBUF, g)

        def pipe_body(t, _):
            for sl in range(IBUF):
                g = IBUF * t + sl
                b = sl % NBUF
                sb = sl % SBUF
                @pl.when(g >= SBUF)
                def _():
                    wait_scatter(sb, (sl + IBUF - SBUF) % IBUF)

                @pl.when(g + 6 < NCHUNK)
                def _():
                    fire_idx((sl + 6) % IBUF, g + 6)

                @pl.when(g + LOOK < NCHUNK)
                def _():
                    wait_idx((sl + LOOK) % IBUF)
                    fire_gather((sl + LOOK) % NBUF, (sl + LOOK) % IBUF)

                wait_gather(b, sl)
                scale(b, sb, sl)
                fire_scatter(sb, sl)
            return _
        lax.fori_loop(0, NCHUNK // IBUF, pipe_body, None)
        for g in range(NCHUNK - SBUF, NCHUNK):
            wait_scatter(g % SBUF, g % IBUF)

    def pack_block():
        def pk(r, _):
            for q in range(DH // 32):
                a = sbufs[0][r, pl.ds(q * 32, 16)]
                bb = sbufs[0][r, pl.ds(q * 32 + 16, 16)]
                gbufs[0][r, pl.ds(q * 32, 32)] = plsc.pack(
                    a, bb, format=plsc.PackFormat.INTERLEAVED)
            return _
        lax.fori_loop(0, RB, pk, None)

    spmm(embbf_hbm)
    plsc.subcore_barrier()
    for b in range(NRB):
        r0 = rbase + b * RB
        pltpu.sync_copy(acc_sh.at[pl.ds(r0, RB), :],
                        sbufs[0].at[pl.ds(0, RB), :])
        pltpu.sync_copy(sbufs[0].at[pl.ds(0, RB), :],
                        tblf_hbm.at[pl.ds(coff + r0, RB), :])
        pack_block()
        pltpu.sync_copy(gbufs[0].at[pl.ds(0, RB), :],
                        tblbf_hbm.at[pl.ds(coff + r0, RB), :])
        for z in range(RB // ZB):
            pltpu.sync_copy(zblk, acc_sh.at[pl.ds(r0 + z * ZB, ZB), :])
    plsc.subcore_barrier()
    spmm(tblbf_hbm)
    plsc.subcore_barrier()
    for b in range(NRB):
        r0 = rbase + b * RB
        pltpu.sync_copy(acc_sh.at[pl.ds(r0, RB), :],
                        sbufs[0].at[pl.ds(0, RB), :])
        pltpu.sync_copy(emb_hbm.at[pl.ds(coff + r0, RB), :],
                        sbufs[1].at[pl.ds(0, RB), :])

        def add_body(r, _):
            for j in range(DH // 16):
                sbufs[0][r, pl.ds(j * 16, 16)] = (
                    sbufs[0][r, pl.ds(j * 16, 16)]
                    + sbufs[1][r, pl.ds(j * 16, 16)])
            return _
        lax.fori_loop(0, RB, add_body, None)
        pltpu.sync_copy(tblf_hbm.at[pl.ds(coff + r0, RB), :],
                        sbufs[1].at[pl.ds(0, RB), :])
        lax.fori_loop(0, RB, add_body, None)
        pltpu.sync_copy(sbufs[0].at[pl.ds(0, RB), :],
                        out_hbm.at[pl.ds(coff + r0, RB), :])


@jax.jit
def kernel(adj_indices, adj_values, uEmbeds, iEmbeds):
    row = adj_indices[0].astype(jnp.int32)
    col = adj_indices[1].astype(jnp.int32)
    pad = E_PAD - N_EDGES
    row = jnp.pad(row, (0, pad))
    col = jnp.pad(col, (0, pad))
    val = jnp.pad(adj_values, (0, pad))

    embeds = jnp.concatenate([uEmbeds, iEmbeds], axis=0)
    emb2 = jnp.zeros((NC * N_PAD, DH), jnp.float32)
    emb2 = emb2.at[:N_NODES].set(embeds[:, :DH])
    emb2 = emb2.at[N_PAD:N_PAD + N_NODES].set(embeds[:, DH:])
    emb_bf = emb2[:, _DIM_ORDER].astype(jnp.bfloat16)

    col2 = jnp.stack([col, col + N_PAD])
    col2 = col2.reshape(NC, E_PAD // 128, 128)
    row2 = row.reshape(E_PAD // 128, 128)
    out2 = _gcn_sc(col2, row2, val, emb_bf, emb2)
    total = jnp.concatenate(
        [out2[:N_NODES], out2[N_PAD:N_PAD + N_NODES]], axis=1)
    return (total[:USER], total[USER:])
